# baseline (speedup 1.0000x reference)
import functools

import jax
import jax.numpy as jnp
from jax import lax
from jax.experimental import pallas as pl
from jax.experimental.pallas import tpu as pltpu, tpu_sc as plsc


def _make_gather_tiled(Vp, D, Bt, H):
    G = 4
    R = 128 * G
    LT = Bt // 128
    n_items = H * (LT // G)
    out_words = Bt * H * D

    info = plsc.get_sparse_core_info()
    NC, NS = info.num_cores, info.num_subcores
    NW = NC * NS
    assert n_items % NW == 0
    per_w = n_items // NW
    NBUF = 4
    assert per_w % NBUF == 0 and per_w >= 2 * NBUF

    mesh = plsc.VectorSubcoreMesh(core_axis_name="c", subcore_axis_name="s")

    @functools.partial(
        pl.kernel,
        mesh=mesh,
        out_type=jax.ShapeDtypeStruct((out_words,), jnp.float32),
        scratch_types=[
            [pltpu.VMEM((R,), jnp.int32) for _ in range(NBUF)],
            [pltpu.VMEM((R, D), jnp.float32) for _ in range(NBUF)],
            [pltpu.VMEM((2 * G * 1024,), jnp.float32) for _ in range(NBUF)],
            [pltpu.SemaphoreType.DMA for _ in range(NBUF)],
            [pltpu.SemaphoreType.DMA for _ in range(NBUF)],
        ],
        compiler_params=pltpu.CompilerParams(
            use_tc_tiling_on_sc=False, needs_layout_passes=False
        ),
    )
    def k(table_hbm, idxt_hbm, out_hbm, idx_v, rows_v, outs_v, sem_g, sem_s):
        wid = lax.axis_index("s") * NC + lax.axis_index("c")
        m_base = wid * per_w

        lanes = lax.iota(jnp.int32, 16)

        def item_coords(m):
            h = m // (LT // G)
            g = m % (LT // G)
            return h, g

        def gather_start(m, b):
            h, g = item_coords(m)
            src = h * Bt + g * R
            pltpu.sync_copy(idxt_hbm.at[pl.ds(src, R)], idx_v[b])
            return pltpu.async_copy(table_hbm.at[idx_v[b]], rows_v[b], sem_g[b])

        def wait_gather(b):
            pltpu.make_async_copy(table_hbm.at[idx_v[b]], rows_v[b], sem_g[b]).wait()

        def shuffle(b):
            def inner(jb, carry):
                for ltl in range(G):
                    j0 = ltl * 128 + jb * 16
                    for t in range(16):
                        dcol = lanes + t
                        dcol = jnp.where(dcol >= 16, dcol - 16, dcol)
                        vals = plsc.load_gather(rows_v[b], [j0 + lanes, dcol])
                        dst = (
                            dcol * 128
                            + jnp.where(dcol >= 8, G * 1024 - 8 * 128, 0)
                            + (ltl * 1024 + jb * 16)
                            + lanes
                        )
                        plsc.store_scatter(outs_v[b], [dst], vals)
                return carry

            lax.fori_loop(0, 8, inner, 0)

        def store_starts(m, b):
            h, g = item_coords(m)
            off0 = (2 * h) * (128 * 1024) + g * (G * 1024)
            nw = G * 1024
            pltpu.async_copy(
                outs_v[b].at[pl.ds(0, nw)], out_hbm.at[pl.ds(off0, nw)], sem_s[b]
            )
            pltpu.async_copy(
                outs_v[b].at[pl.ds(nw, nw)],
                out_hbm.at[pl.ds(off0 + 128 * 1024, nw)],
                sem_s[b],
            )

        def store_waits(m, b):
            h, g = item_coords(m)
            off0 = (2 * h) * (128 * 1024) + g * (G * 1024)
            nw = G * 1024
            pltpu.make_async_copy(
                outs_v[b].at[pl.ds(0, nw)], out_hbm.at[pl.ds(off0, nw)], sem_s[b]
            ).wait()
            pltpu.make_async_copy(
                outs_v[b].at[pl.ds(nw, nw)],
                out_hbm.at[pl.ds(off0 + 128 * 1024, nw)],
                sem_s[b],
            ).wait()

        for b in range(NBUF):
            gather_start(m_base + b, b)

        def body(o, carry):
            m0 = m_base + o * NBUF
            for b in range(NBUF):
                m = m0 + b
                wait_gather(b)

                @pl.when(o > 0)
                def _():
                    store_waits(m - NBUF, b)

                shuffle(b)
                store_starts(m, b)

                @pl.when(m + NBUF < m_base + per_w)
                def _():
                    gather_start(m + NBUF, b)

            return carry

        lax.fori_loop(0, per_w // NBUF, body, 0)

        for b in range(NBUF):
            store_waits(m_base + per_w - NBUF + b, b)

    return k


def kernel(x, table):
    Bt, H = x.shape
    V, D = table.shape
    B = Bt * H
    idxT = x.T.reshape(B)
    flat = _make_gather_tiled(V, D, Bt, H)(table, idxT)
    f5 = flat.reshape(H, 2, Bt // 128, 8, 128)
    return jnp.transpose(f5, (2, 4, 0, 1, 3)).reshape(Bt, H, D)


# speedup vs baseline: 1.1459x; 1.1459x over previous
import functools

import jax
import jax.numpy as jnp
from jax import lax
from jax.experimental import pallas as pl
from jax.experimental.pallas import tpu as pltpu, tpu_sc as plsc


def _make_gather_tiled(Vp, D, Bt, H):
    G = 8
    R = 128 * G
    LT = Bt // 128
    n_items = H * (LT // G)
    out_words = Bt * H * D

    info = plsc.get_sparse_core_info()
    NC, NS = info.num_cores, info.num_subcores
    NW = NC * NS
    assert n_items % NW == 0
    per_w = n_items // NW
    NBUF = 2
    assert per_w % NBUF == 0 and per_w >= 2 * NBUF

    mesh = plsc.VectorSubcoreMesh(core_axis_name="c", subcore_axis_name="s")

    @functools.partial(
        pl.kernel,
        mesh=mesh,
        out_type=jax.ShapeDtypeStruct((out_words,), jnp.float32),
        scratch_types=[
            [pltpu.VMEM((R,), jnp.int32) for _ in range(NBUF)],
            [pltpu.VMEM((R, D), jnp.float32) for _ in range(NBUF)],
            [pltpu.VMEM((2 * G * 1024,), jnp.float32) for _ in range(NBUF)],
            [pltpu.SemaphoreType.DMA for _ in range(NBUF)],
            [pltpu.SemaphoreType.DMA for _ in range(NBUF)],
        ],
        compiler_params=pltpu.CompilerParams(
            use_tc_tiling_on_sc=False, needs_layout_passes=False
        ),
    )
    def k(table_hbm, idxt_hbm, out_hbm, idx_v, rows_v, outs_v, sem_g, sem_s):
        wid = lax.axis_index("s") * NC + lax.axis_index("c")
        m_base = wid * per_w

        lanes = lax.iota(jnp.int32, 16)

        def item_coords(m):
            h = m // (LT // G)
            g = m % (LT // G)
            return h, g

        def gather_start(m, b):
            h, g = item_coords(m)
            src = h * Bt + g * R
            pltpu.sync_copy(idxt_hbm.at[pl.ds(src, R)], idx_v[b])
            return pltpu.async_copy(table_hbm.at[idx_v[b]], rows_v[b], sem_g[b])

        def wait_gather(b):
            pltpu.make_async_copy(table_hbm.at[idx_v[b]], rows_v[b], sem_g[b]).wait()

        def shuffle(b):
            def inner(jb, carry):
                for ltl in range(G):
                    j0 = ltl * 128 + jb * 16
                    for t in range(16):
                        dcol = lanes + t
                        dcol = jnp.where(dcol >= 16, dcol - 16, dcol)
                        vals = plsc.load_gather(rows_v[b], [j0 + lanes, dcol])
                        dst = (
                            dcol * 128
                            + jnp.where(dcol >= 8, G * 1024 - 8 * 128, 0)
                            + (ltl * 1024 + jb * 16)
                            + lanes
                        )
                        plsc.store_scatter(outs_v[b], [dst], vals)
                return carry

            lax.fori_loop(0, 8, inner, 0)

        def store_starts(m, b):
            h, g = item_coords(m)
            off0 = (2 * h) * (128 * 1024) + g * (G * 1024)
            nw = G * 1024
            pltpu.async_copy(
                outs_v[b].at[pl.ds(0, nw)], out_hbm.at[pl.ds(off0, nw)], sem_s[b]
            )
            pltpu.async_copy(
                outs_v[b].at[pl.ds(nw, nw)],
                out_hbm.at[pl.ds(off0 + 128 * 1024, nw)],
                sem_s[b],
            )

        def store_waits(m, b):
            h, g = item_coords(m)
            off0 = (2 * h) * (128 * 1024) + g * (G * 1024)
            nw = G * 1024
            pltpu.make_async_copy(
                outs_v[b].at[pl.ds(0, nw)], out_hbm.at[pl.ds(off0, nw)], sem_s[b]
            ).wait()
            pltpu.make_async_copy(
                outs_v[b].at[pl.ds(nw, nw)],
                out_hbm.at[pl.ds(off0 + 128 * 1024, nw)],
                sem_s[b],
            ).wait()

        for b in range(NBUF):
            gather_start(m_base + b, b)

        def body(o, carry):
            m0 = m_base + o * NBUF
            for b in range(NBUF):
                m = m0 + b
                wait_gather(b)

                @pl.when(o > 0)
                def _():
                    store_waits(m - NBUF, b)

                shuffle(b)
                store_starts(m, b)

                @pl.when(m + NBUF < m_base + per_w)
                def _():
                    gather_start(m + NBUF, b)

            return carry

        lax.fori_loop(0, per_w // NBUF, body, 0)

        for b in range(NBUF):
            store_waits(m_base + per_w - NBUF + b, b)

    return k


def kernel(x, table):
    Bt, H = x.shape
    V, D = table.shape
    B = Bt * H
    idxT = x.T.reshape(B)
    flat = _make_gather_tiled(V, D, Bt, H)(table, idxT)
    f5 = flat.reshape(H, 2, Bt // 128, 8, 128)
    return jnp.transpose(f5, (2, 4, 0, 1, 3)).reshape(Bt, H, D)
